# baseline (speedup 1.0000x reference)
import functools

import jax
import jax.numpy as jnp
from jax import lax
from jax.experimental import pallas as pl
from jax.experimental.pallas import tpu as pltpu
from jax.experimental.pallas import tpu_sc as plsc

N = 50000
E = 800000
B = 16
C = 10
CP = 16
NP = 50176
NC, NS = 2, 16
NW = NC * NS
EPT = 25088
EP = NW * EPT
KC = 3136
SLICE = NP // NS
BLK = 6272
NBLK = NP // BLK

_mesh = plsc.VectorSubcoreMesh(
    core_axis_name="c", subcore_axis_name="s", num_cores=NC, num_subcores=NS
)


def _deg_body(dst_hbm, w_hbm, deg_out, idx_all, w_all, zbuf, sem, deg_sh):
    c = lax.axis_index("c")
    s = lax.axis_index("s")
    wid = s * NC + c
    base = wid * EPT

    d_idx = pltpu.async_copy(dst_hbm.at[pl.ds(base, EPT)], idx_all, sem)
    d_w = pltpu.async_copy(w_hbm.at[pl.ds(base, EPT)], w_all, sem)

    def _z(i, carry):
        zbuf[pl.ds(i * 16, 16)] = jnp.zeros((16,), jnp.float32)
        return carry

    lax.fori_loop(0, SLICE // 16, _z, 0)
    pltpu.sync_copy(zbuf, deg_sh.at[pl.ds(s * SLICE, SLICE)])
    plsc.subcore_barrier()

    d_idx.wait()
    d_w.wait()
    pltpu.sync_copy(w_all, deg_sh.at[idx_all], add=True)
    plsc.subcore_barrier()

    pltpu.sync_copy(deg_sh.at[pl.ds(s * SLICE, SLICE)], zbuf)
    pltpu.sync_copy(zbuf, deg_out.at[pl.ds(c * NP + s * SLICE, SLICE)])


_deg_call = functools.partial(
    pl.kernel,
    out_type=jax.ShapeDtypeStruct((NC * NP,), jnp.float32),
    mesh=_mesh,
    scratch_types=[
        pltpu.VMEM((EPT,), jnp.int32),
        pltpu.VMEM((EPT,), jnp.float32),
        pltpu.VMEM((SLICE,), jnp.float32),
        pltpu.SemaphoreType.DMA,
        pltpu.VMEM_SHARED((NP,), jnp.float32),
    ],
)(_deg_body)


NPT = NP // NW


def _prep_body(deg_hbm, fcwt_hbm, dis_out, u_out, d0, d1, fw, disb, sem):
    c = lax.axis_index("c")
    s = lax.axis_index("s")
    wid = s * NC + c
    bn = wid * NPT
    cp0 = pltpu.async_copy(deg_hbm.at[pl.ds(bn, NPT)], d0, sem)
    cp1 = pltpu.async_copy(deg_hbm.at[pl.ds(NP + bn, NPT)], d1, sem)
    cpf = pltpu.async_copy(fcwt_hbm.at[pl.ds(bn, NPT)], fw, sem)
    cp0.wait()
    cp1.wait()
    cpf.wait()

    def _n(j, carry):
        d = d0[pl.ds(j * 16, 16)] + d1[pl.ds(j * 16, 16)] + 1.0
        i = plsc.bitcast(d, jnp.int32)
        y = plsc.bitcast(jnp.int32(0x5F3759DF) - (i >> 1), jnp.float32)
        y = y * (1.5 - 0.5 * d * y * y)
        y = y * (1.5 - 0.5 * d * y * y)
        y = y * (1.5 - 0.5 * d * y * y)
        disb[pl.ds(j * 16, 16)] = y
        for t in range(16):
            k = j * 16 + t
            fw[k] = fw[k] * y[t]
        return carry

    lax.fori_loop(0, NPT // 16, _n, 0)
    pltpu.sync_copy(disb, dis_out.at[pl.ds(bn, NPT)])
    pltpu.sync_copy(fw, u_out.at[pl.ds(bn, NPT)])


_prep_call = functools.partial(
    pl.kernel,
    out_type=[
        jax.ShapeDtypeStruct((NP,), jnp.float32),
        jax.ShapeDtypeStruct((NP, CP), jnp.float32),
    ],
    mesh=_mesh,
    scratch_types=[
        pltpu.VMEM((NPT,), jnp.float32),
        pltpu.VMEM((NPT,), jnp.float32),
        pltpu.VMEM((NPT, CP), jnp.float32),
        pltpu.VMEM((NPT,), jnp.float32),
        pltpu.SemaphoreType.DMA,
    ],
    compiler_params=pltpu.CompilerParams(use_tc_tiling_on_sc=False, needs_layout_passes=False),
)(_prep_body)


KCM = 784
NCH = EPT // KCM
ND = 4
NSL = SLICE // KCM


def _msg_body(e3_hbm, u_hbm, v_out,
              e0, e1, e2, e3,
              rows0, rows1, rows2, rows3, tbuf,
              st0, st1, st2, st3, sg0, sg1, sg2, sg3,
              ss0, ss1, ss2, ss3, swo, v_sh):
    c = lax.axis_index("c")
    s = lax.axis_index("s")
    wid = s * NC + c
    rbase = wid * NCH

    ebuf = (e0, e1, e2, e3)
    rows = (rows0, rows1, rows2, rows3)
    stsem = (st0, st1, st2, st3)
    gsem = (sg0, sg1, sg2, sg3)
    ssem = (ss0, ss1, ss2, ss3)

    def _stage(i):
        r = i % ND
        return pltpu.async_copy(e3_hbm.at[i + rbase], ebuf[r], stsem[r])

    def _gather(i):
        r = i % ND
        return pltpu.async_copy(u_hbm.at[ebuf[r].at[0]], rows[r], gsem[r])

    std = [None] * ND
    sd = [None] * ND
    std[0] = _stage(0)
    std[1] = _stage(1)
    std[2] = _stage(2)

    def _z(i, carry):
        rows0[i] = jnp.zeros((CP,), jnp.float32)
        return carry

    lax.fori_loop(0, KCM, _z, 0)
    for j in range(NSL):
        pltpu.sync_copy(rows0, v_sh.at[pl.ds(s * SLICE + j * KCM, KCM)])
    plsc.subcore_barrier()

    gd = [None] * ND
    for j in range(2):
        std[j].wait()
        gd[j] = _gather(j)
    for i in range(NCH):
        r = i % ND
        if i >= 1:
            sd[(i - 1) % ND].wait()
        if i + 3 < NCH:
            std[(i + 3) % ND] = _stage(i + 3)
        if i + 2 < NCH:
            std[(i + 2) % ND].wait()
            gd[(i + 2) % ND] = _gather(i + 2)
        gd[r].wait()

        def _scale(j, carry, r=r):
            wvec = plsc.bitcast(ebuf[r][2, pl.ds(j * 16, 16)], jnp.float32)
            for t in range(16):
                k = j * 16 + t
                rows[r][k] = rows[r][k] * wvec[t]
            return carry

        lax.fori_loop(0, KCM // 16, _scale, 0)
        sd[r] = pltpu.async_copy(rows[r], v_sh.at[ebuf[r].at[1]], ssem[r],
                                 add=True)
    sd[(NCH - 1) % ND].wait()
    plsc.subcore_barrier()

    lanes = lax.iota(jnp.int32, 16)
    wod = []
    for j in range(NSL):
        buf = rows[j % ND]
        pltpu.sync_copy(v_sh.at[pl.ds(s * SLICE + j * KCM, KCM)], buf)
        for cls in range(CP):
            cvec = jnp.full((16,), cls, jnp.int32)

            def _t(q, carry, buf=buf, cls=cls, cvec=cvec):
                g = plsc.load_gather(buf, [q * 16 + lanes, cvec])
                tbuf[cls, pl.ds(q * 16, 16)] = g
                return carry

            lax.fori_loop(0, KCM // 16, _t, 0)
        for cls in range(CP):
            wod.append(pltpu.async_copy(
                tbuf.at[cls],
                v_out.at[c * CP + cls, pl.ds(s * SLICE + j * KCM, KCM)],
                swo))
        for d in wod:
            d.wait()
        wod = []


_msg_call = functools.partial(
    pl.kernel,
    out_type=jax.ShapeDtypeStruct((NC * CP, NP), jnp.float32),
    mesh=_mesh,
    scratch_types=(
        [pltpu.VMEM((3, KCM), jnp.int32)] * 4
        + [pltpu.VMEM((KCM, CP), jnp.float32)] * 4
        + [pltpu.VMEM((CP, KCM), jnp.float32)]
        + [pltpu.SemaphoreType.DMA] * 13
        + [pltpu.VMEM_SHARED((NP, CP), jnp.float32)]
    ),
    compiler_params=pltpu.CompilerParams(use_tc_tiling_on_sc=False, needs_layout_passes=False),
)(_msg_body)


def _final_body(x_ref, vt_ref, fcw_ref, dis_ref, b_ref, out_ref, acc):
    k = pl.program_id(0)

    @pl.when(k == 0)
    def _init():
        acc[...] = jnp.zeros_like(acc)

    vt = vt_ref[...]
    dis = dis_ref[...]
    z = dis * (vt[0] + vt[1] + dis * fcw_ref[...])
    acc[...] += lax.dot_general(
        x_ref[...], z, (((1,), (1,)), ((), ())),
        precision=lax.Precision.HIGHEST,
        preferred_element_type=jnp.float32)

    @pl.when(k == NBLK - 1)
    def _fin():
        logits = acc[...] + b_ref[...]
        m = jnp.max(logits, axis=1, keepdims=True)
        e = jnp.exp(logits - m)
        out_ref[...] = (e / jnp.sum(e, axis=1, keepdims=True))[:, :C]


def _final_call(xp, vt3, fcw_pad, dis_row, bp):
    return pl.pallas_call(
        _final_body,
        grid=(NBLK,),
        in_specs=[
            pl.BlockSpec((B, BLK), lambda k: (0, k)),
            pl.BlockSpec((NC, CP, BLK), lambda k: (0, 0, k)),
            pl.BlockSpec((CP, BLK), lambda k: (0, k)),
            pl.BlockSpec((1, BLK), lambda k: (0, k)),
            pl.BlockSpec((1, CP), lambda k: (0, 0)),
        ],
        out_specs=pl.BlockSpec((B, C), lambda k: (0, 0)),
        out_shape=jax.ShapeDtypeStruct((B, C), jnp.float32),
        scratch_shapes=[pltpu.VMEM((B, CP), jnp.float32)],
    )(xp, vt3, fcw_pad, dis_row, bp)


def kernel(x, edge_index, edge_weight, fc_w, fc_b):
    src = edge_index[0].astype(jnp.int32)
    dst = edge_index[1].astype(jnp.int32)
    pad_e = EP - E
    pad_idx = jnp.arange(pad_e, dtype=jnp.int32)
    src_p = jnp.concatenate([src, pad_idx])
    dst_p = jnp.concatenate([dst, pad_idx])
    w_p = jnp.concatenate([edge_weight, jnp.zeros((pad_e,), jnp.float32)])

    fcwt = jnp.pad(fc_w, ((0, CP - C), (0, NP - N))).T
    xp = jnp.pad(x, ((0, 0), (0, NP - N)))
    bp = jnp.concatenate(
        [fc_b, jnp.full((CP - C,), -1e30, jnp.float32)]
    ).reshape(1, CP)

    deg_flat = _deg_call(dst_p, w_p)
    dis_lin, u = _prep_call(deg_flat, fcwt)
    e3 = jnp.concatenate(
        [dst_p.reshape(EP // KCM, 1, KCM),
         src_p.reshape(EP // KCM, 1, KCM),
         lax.bitcast_convert_type(w_p, jnp.int32).reshape(EP // KCM, 1, KCM)],
        axis=1)
    vt = _msg_call(e3, u)
    vt3 = vt.reshape(NC, CP, NP)
    dis_row = dis_lin.reshape(1, NP)
    fcw_pad = jnp.pad(fc_w, ((0, CP - C), (0, NP - N)))
    return _final_call(xp, vt3, fcw_pad, dis_row, bp)


# speedup vs baseline: 1.4267x; 1.4267x over previous
import functools

import jax
import jax.numpy as jnp
from jax import lax
from jax.experimental import pallas as pl
from jax.experimental.pallas import tpu as pltpu
from jax.experimental.pallas import tpu_sc as plsc

N = 50000
E = 800000
B = 16
C = 10
CP = 16
NP = 50176
NC, NS = 2, 16
NW = NC * NS
EPT = 25088
EP = NW * EPT
KC = 3136
SLICE = NP // NS
BLK = 6272
NBLK = NP // BLK

_mesh = plsc.VectorSubcoreMesh(
    core_axis_name="c", subcore_axis_name="s", num_cores=NC, num_subcores=NS
)


def _deg_body(dst_hbm, w_hbm, deg_out, idx_all, w_all, zbuf, sem, deg_sh):
    c = lax.axis_index("c")
    s = lax.axis_index("s")
    wid = s * NC + c
    base = wid * EPT

    d_idx = pltpu.async_copy(dst_hbm.at[pl.ds(base, EPT)], idx_all, sem)
    d_w = pltpu.async_copy(w_hbm.at[pl.ds(base, EPT)], w_all, sem)

    def _z(i, carry):
        zbuf[pl.ds(i * 16, 16)] = jnp.zeros((16,), jnp.float32)
        return carry

    lax.fori_loop(0, SLICE // 16, _z, 0)
    pltpu.sync_copy(zbuf, deg_sh.at[pl.ds(s * SLICE, SLICE)])
    plsc.subcore_barrier()

    d_idx.wait()
    d_w.wait()
    pltpu.sync_copy(w_all, deg_sh.at[idx_all], add=True)
    plsc.subcore_barrier()

    pltpu.sync_copy(deg_sh.at[pl.ds(s * SLICE, SLICE)], zbuf)
    pltpu.sync_copy(zbuf, deg_out.at[pl.ds(c * NP + s * SLICE, SLICE)])


_deg_call = functools.partial(
    pl.kernel,
    out_type=jax.ShapeDtypeStruct((NC * NP,), jnp.float32),
    mesh=_mesh,
    scratch_types=[
        pltpu.VMEM((EPT,), jnp.int32),
        pltpu.VMEM((EPT,), jnp.float32),
        pltpu.VMEM((SLICE,), jnp.float32),
        pltpu.SemaphoreType.DMA,
        pltpu.VMEM_SHARED((NP,), jnp.float32),
    ],
)(_deg_body)


NPT = NP // NW


def _prep_body(deg_hbm, fcwt_hbm, dis_out, u_out, d0, d1, fw, disb, sem):
    c = lax.axis_index("c")
    s = lax.axis_index("s")
    wid = s * NC + c
    bn = wid * NPT
    cp0 = pltpu.async_copy(deg_hbm.at[pl.ds(bn, NPT)], d0, sem)
    cp1 = pltpu.async_copy(deg_hbm.at[pl.ds(NP + bn, NPT)], d1, sem)
    cpf = pltpu.async_copy(fcwt_hbm.at[pl.ds(bn, NPT)], fw, sem)
    cp0.wait()
    cp1.wait()
    cpf.wait()

    def _n(j, carry):
        d = d0[pl.ds(j * 16, 16)] + d1[pl.ds(j * 16, 16)] + 1.0
        i = plsc.bitcast(d, jnp.int32)
        y = plsc.bitcast(jnp.int32(0x5F3759DF) - (i >> 1), jnp.float32)
        y = y * (1.5 - 0.5 * d * y * y)
        y = y * (1.5 - 0.5 * d * y * y)
        y = y * (1.5 - 0.5 * d * y * y)
        disb[pl.ds(j * 16, 16)] = y
        for t in range(16):
            k = j * 16 + t
            fw[k] = fw[k] * y[t]
        return carry

    lax.fori_loop(0, NPT // 16, _n, 0)
    pltpu.sync_copy(disb, dis_out.at[pl.ds(bn, NPT)])
    pltpu.sync_copy(fw, u_out.at[pl.ds(bn, NPT)])


_prep_call = functools.partial(
    pl.kernel,
    out_type=[
        jax.ShapeDtypeStruct((NP,), jnp.float32),
        jax.ShapeDtypeStruct((NP, CP), jnp.float32),
    ],
    mesh=_mesh,
    scratch_types=[
        pltpu.VMEM((NPT,), jnp.float32),
        pltpu.VMEM((NPT,), jnp.float32),
        pltpu.VMEM((NPT, CP), jnp.float32),
        pltpu.VMEM((NPT,), jnp.float32),
        pltpu.SemaphoreType.DMA,
    ],
    compiler_params=pltpu.CompilerParams(use_tc_tiling_on_sc=False, needs_layout_passes=False),
)(_prep_body)


KCM = 784
NCH = EPT // KCM
ND = 4
NSL = SLICE // KCM


def _msg_body(src2_hbm, dst2_hbm, w2_hbm, u_hbm, v_out,
              idxs0, idxs1, idxs2, idxs3, idxd0, idxd1, idxd2, idxd3,
              w0, w1, w2, w3,
              rows0, rows1, rows2, rows3, tbuf,
              st0, st1, st2, st3, sg0, sg1, sg2, sg3,
              ss0, ss1, ss2, ss3, swo, v_sh):
    c = lax.axis_index("c")
    s = lax.axis_index("s")
    wid = s * NC + c
    rbase = wid * NCH

    idxs = (idxs0, idxs1, idxs2, idxs3)
    idxd = (idxd0, idxd1, idxd2, idxd3)
    wv = (w0, w1, w2, w3)
    rows = (rows0, rows1, rows2, rows3)
    stsem = (st0, st1, st2, st3)
    gsem = (sg0, sg1, sg2, sg3)
    ssem = (ss0, ss1, ss2, ss3)

    def _stage(i):
        r = i % ND
        return (pltpu.async_copy(dst2_hbm.at[i + rbase], idxd[r], stsem[r]),
                pltpu.async_copy(src2_hbm.at[i + rbase], idxs[r], stsem[r]),
                pltpu.async_copy(w2_hbm.at[i + rbase], wv[r], stsem[r]))

    def _gather(i):
        r = i % ND
        return pltpu.async_copy(u_hbm.at[idxd[r]], rows[r], gsem[r])

    std = [None] * ND
    sd = [None] * ND
    std[0] = _stage(0)
    std[1] = _stage(1)
    std[2] = _stage(2)

    def _z(i, carry):
        rows0[i] = jnp.zeros((CP,), jnp.float32)
        return carry

    lax.fori_loop(0, KCM, _z, 0)
    for j in range(NSL):
        pltpu.sync_copy(rows0, v_sh.at[pl.ds(s * SLICE + j * KCM, KCM)])
    plsc.subcore_barrier()

    gd = [None] * ND
    for j in range(2):
        for d in std[j]:
            d.wait()
        gd[j] = _gather(j)
    for i in range(NCH):
        r = i % ND
        if i >= 1:
            sd[(i - 1) % ND].wait()
        if i + 3 < NCH:
            std[(i + 3) % ND] = _stage(i + 3)
        if i + 2 < NCH:
            for d in std[(i + 2) % ND]:
                d.wait()
            gd[(i + 2) % ND] = _gather(i + 2)
        gd[r].wait()

        def _scale(j, carry, r=r):
            wvec = wv[r][pl.ds(j * 16, 16)]
            for t in range(16):
                k = j * 16 + t
                rows[r][k] = rows[r][k] * wvec[t]
            return carry

        lax.fori_loop(0, KCM // 16, _scale, 0)
        sd[r] = pltpu.async_copy(rows[r], v_sh.at[idxs[r]], ssem[r],
                                 add=True)
    sd[(NCH - 1) % ND].wait()
    plsc.subcore_barrier()

    lanes = lax.iota(jnp.int32, 16)
    wod = []
    for j in range(NSL):
        buf = rows[j % ND]
        pltpu.sync_copy(v_sh.at[pl.ds(s * SLICE + j * KCM, KCM)], buf)
        for cls in range(CP):
            cvec = jnp.full((16,), cls, jnp.int32)

            def _t(q, carry, buf=buf, cls=cls, cvec=cvec):
                g = plsc.load_gather(buf, [q * 16 + lanes, cvec])
                tbuf[cls, pl.ds(q * 16, 16)] = g
                return carry

            lax.fori_loop(0, KCM // 16, _t, 0)
        for cls in range(CP):
            wod.append(pltpu.async_copy(
                tbuf.at[cls],
                v_out.at[c * CP + cls, pl.ds(s * SLICE + j * KCM, KCM)],
                swo))
        for d in wod:
            d.wait()
        wod = []


_msg_call = functools.partial(
    pl.kernel,
    out_type=jax.ShapeDtypeStruct((NC * CP, NP), jnp.float32),
    mesh=_mesh,
    scratch_types=(
        [pltpu.VMEM((KCM,), jnp.int32)] * 8
        + [pltpu.VMEM((KCM,), jnp.float32)] * 4
        + [pltpu.VMEM((KCM, CP), jnp.float32)] * 4
        + [pltpu.VMEM((CP, KCM), jnp.float32)]
        + [pltpu.SemaphoreType.DMA] * 13
        + [pltpu.VMEM_SHARED((NP, CP), jnp.float32)]
    ),
    compiler_params=pltpu.CompilerParams(use_tc_tiling_on_sc=False, needs_layout_passes=False),
)(_msg_body)


def _final_body(x_ref, vt_ref, fcw_ref, dis_ref, b_ref, out_ref, acc):
    k = pl.program_id(0)

    @pl.when(k == 0)
    def _init():
        acc[...] = jnp.zeros_like(acc)

    vt = vt_ref[...]
    dis = dis_ref[...]
    z = dis * (vt[0] + vt[1] + dis * fcw_ref[...])
    acc[...] += lax.dot_general(
        x_ref[...], z, (((1,), (1,)), ((), ())),
        precision=lax.Precision.HIGHEST,
        preferred_element_type=jnp.float32)

    @pl.when(k == NBLK - 1)
    def _fin():
        logits = acc[...] + b_ref[...]
        m = jnp.max(logits, axis=1, keepdims=True)
        e = jnp.exp(logits - m)
        out_ref[...] = (e / jnp.sum(e, axis=1, keepdims=True))[:, :C]


def _final_call(xp, vt3, fcw_pad, dis_row, bp):
    return pl.pallas_call(
        _final_body,
        grid=(NBLK,),
        in_specs=[
            pl.BlockSpec((B, BLK), lambda k: (0, k)),
            pl.BlockSpec((NC, CP, BLK), lambda k: (0, 0, k)),
            pl.BlockSpec((CP, BLK), lambda k: (0, k)),
            pl.BlockSpec((1, BLK), lambda k: (0, k)),
            pl.BlockSpec((1, CP), lambda k: (0, 0)),
        ],
        out_specs=pl.BlockSpec((B, C), lambda k: (0, 0)),
        out_shape=jax.ShapeDtypeStruct((B, C), jnp.float32),
        scratch_shapes=[pltpu.VMEM((B, CP), jnp.float32)],
    )(xp, vt3, fcw_pad, dis_row, bp)


def kernel(x, edge_index, edge_weight, fc_w, fc_b):
    ei3 = edge_index.astype(jnp.int32).reshape(2, E // 128, 128)
    ei3 = ei3.transpose(1, 0, 2)
    src = ei3[:, 0, :].reshape(E)
    dst = ei3[:, 1, :].reshape(E)
    pad_e = EP - E
    pad_idx = jnp.arange(pad_e, dtype=jnp.int32)
    src_p = jnp.concatenate([src, pad_idx])
    dst_p = jnp.concatenate([dst, pad_idx])
    w_p = jnp.concatenate([edge_weight, jnp.zeros((pad_e,), jnp.float32)])

    fcwt = jnp.pad(fc_w, ((0, CP - C), (0, NP - N))).T
    xp = jnp.pad(x, ((0, 0), (0, NP - N)))
    bp = jnp.concatenate(
        [fc_b, jnp.full((CP - C,), -1e30, jnp.float32)]
    ).reshape(1, CP)

    deg_flat = _deg_call(dst_p, w_p)
    dis_lin, u = _prep_call(deg_flat, fcwt)
    src2 = src_p.reshape(EP // KCM, KCM)
    dst2 = dst_p.reshape(EP // KCM, KCM)
    w2 = w_p.reshape(EP // KCM, KCM)
    vt = _msg_call(src2, dst2, w2, u)
    vt3 = vt.reshape(NC, CP, NP)
    dis_row = dis_lin.reshape(1, NP)
    fcw_pad = jnp.pad(fc_w, ((0, CP - C), (0, NP - N)))
    return _final_call(xp, vt3, fcw_pad, dis_row, bp)


# speedup vs baseline: 1.4268x; 1.0001x over previous
import functools

import jax
import jax.numpy as jnp
from jax import lax
from jax.experimental import pallas as pl
from jax.experimental.pallas import tpu as pltpu
from jax.experimental.pallas import tpu_sc as plsc

N = 50000
E = 800000
B = 16
C = 10
CP = 16
NP = 50176
NC, NS = 2, 16
NW = NC * NS
EPT = 25088
EP = NW * EPT
KC = 3136
SLICE = NP // NS
BLK = 6272
NBLK = NP // BLK

_mesh = plsc.VectorSubcoreMesh(
    core_axis_name="c", subcore_axis_name="s", num_cores=NC, num_subcores=NS
)


def _deg_body(dst_hbm, w_hbm, deg_out, idx_all, w_all, zbuf, sem, deg_sh):
    c = lax.axis_index("c")
    s = lax.axis_index("s")
    wid = s * NC + c
    base = wid * EPT

    d_idx = pltpu.async_copy(dst_hbm.at[pl.ds(base, EPT)], idx_all, sem)
    d_w = pltpu.async_copy(w_hbm.at[pl.ds(base, EPT)], w_all, sem)

    def _z(i, carry):
        zbuf[pl.ds(i * 16, 16)] = jnp.zeros((16,), jnp.float32)
        return carry

    lax.fori_loop(0, SLICE // 16, _z, 0)
    pltpu.sync_copy(zbuf, deg_sh.at[pl.ds(s * SLICE, SLICE)])
    plsc.subcore_barrier()

    d_idx.wait()
    d_w.wait()
    pltpu.sync_copy(w_all, deg_sh.at[idx_all], add=True)
    plsc.subcore_barrier()

    pltpu.sync_copy(deg_sh.at[pl.ds(s * SLICE, SLICE)], zbuf)
    pltpu.sync_copy(zbuf, deg_out.at[pl.ds(c * NP + s * SLICE, SLICE)])


_deg_call = functools.partial(
    pl.kernel,
    out_type=jax.ShapeDtypeStruct((NC * NP,), jnp.float32),
    mesh=_mesh,
    scratch_types=[
        pltpu.VMEM((EPT,), jnp.int32),
        pltpu.VMEM((EPT,), jnp.float32),
        pltpu.VMEM((SLICE,), jnp.float32),
        pltpu.SemaphoreType.DMA,
        pltpu.VMEM_SHARED((NP,), jnp.float32),
    ],
)(_deg_body)


NPT = NP // NW


def _prep_body(deg_hbm, fcwt_hbm, dis_out, u_out, d0, d1, fw, disb, sem):
    c = lax.axis_index("c")
    s = lax.axis_index("s")
    wid = s * NC + c
    bn = wid * NPT
    cp0 = pltpu.async_copy(deg_hbm.at[pl.ds(bn, NPT)], d0, sem)
    cp1 = pltpu.async_copy(deg_hbm.at[pl.ds(NP + bn, NPT)], d1, sem)
    cpf = pltpu.async_copy(fcwt_hbm.at[pl.ds(bn * CP, NPT * CP)], fw, sem)
    cp0.wait()
    cp1.wait()
    cpf.wait()

    def _n(j, carry):
        d = d0[pl.ds(j * 16, 16)] + d1[pl.ds(j * 16, 16)] + 1.0
        i = plsc.bitcast(d, jnp.int32)
        y = plsc.bitcast(jnp.int32(0x5F3759DF) - (i >> 1), jnp.float32)
        y = y * (1.5 - 0.5 * d * y * y)
        y = y * (1.5 - 0.5 * d * y * y)
        y = y * (1.5 - 0.5 * d * y * y)
        disb[pl.ds(j * 16, 16)] = y
        for t in range(16):
            k = j * 16 + t
            fw[pl.ds(k * CP, CP)] = fw[pl.ds(k * CP, CP)] * y[t]
        return carry

    lax.fori_loop(0, NPT // 16, _n, 0)
    pltpu.sync_copy(disb, dis_out.at[pl.ds(bn, NPT)])
    pltpu.sync_copy(fw, u_out.at[pl.ds(bn * CP, NPT * CP)])


_prep_call = functools.partial(
    pl.kernel,
    out_type=[
        jax.ShapeDtypeStruct((NP,), jnp.float32),
        jax.ShapeDtypeStruct((NP * CP,), jnp.float32),
    ],
    mesh=_mesh,
    scratch_types=[
        pltpu.VMEM((NPT,), jnp.float32),
        pltpu.VMEM((NPT,), jnp.float32),
        pltpu.VMEM((NPT * CP,), jnp.float32),
        pltpu.VMEM((NPT,), jnp.float32),
        pltpu.SemaphoreType.DMA,
    ],
    compiler_params=pltpu.CompilerParams(use_tc_tiling_on_sc=False, needs_layout_passes=False),
)(_prep_body)


KCM = 784
NCH = EPT // KCM
ND = 4
NSL = SLICE // KCM


def _msg_body(src2_hbm, dst2_hbm, w2_hbm, u_hbm, v_out,
              idxs0, idxs1, idxs2, idxs3, idxd0, idxd1, idxd2, idxd3,
              w0, w1, w2, w3,
              rows0, rows1, rows2, rows3, tbuf,
              st0, st1, st2, st3, sg0, sg1, sg2, sg3,
              ss0, ss1, ss2, ss3, swo, v_sh):
    c = lax.axis_index("c")
    s = lax.axis_index("s")
    wid = s * NC + c
    rbase = wid * NCH

    idxs = (idxs0, idxs1, idxs2, idxs3)
    idxd = (idxd0, idxd1, idxd2, idxd3)
    wv = (w0, w1, w2, w3)
    rows = (rows0, rows1, rows2, rows3)
    stsem = (st0, st1, st2, st3)
    gsem = (sg0, sg1, sg2, sg3)
    ssem = (ss0, ss1, ss2, ss3)

    def _stage(i):
        r = i % ND
        return (pltpu.async_copy(dst2_hbm.at[i + rbase], idxd[r], stsem[r]),
                pltpu.async_copy(src2_hbm.at[i + rbase], idxs[r], stsem[r]),
                pltpu.async_copy(w2_hbm.at[i + rbase], wv[r], stsem[r]))

    def _gather(i):
        r = i % ND
        return pltpu.async_copy(u_hbm.at[idxd[r]], rows[r], gsem[r])

    std = [None] * ND
    sd = [None] * ND
    std[0] = _stage(0)
    std[1] = _stage(1)
    std[2] = _stage(2)

    def _z(i, carry):
        rows0[i] = jnp.zeros((CP,), jnp.float32)
        return carry

    lax.fori_loop(0, KCM, _z, 0)
    for j in range(NSL):
        pltpu.sync_copy(rows0, v_sh.at[pl.ds(s * SLICE + j * KCM, KCM)])
    plsc.subcore_barrier()

    gd = [None] * ND
    for j in range(2):
        for d in std[j]:
            d.wait()
        gd[j] = _gather(j)
    for i in range(NCH):
        r = i % ND
        if i >= 1:
            sd[(i - 1) % ND].wait()
        if i + 3 < NCH:
            std[(i + 3) % ND] = _stage(i + 3)
        if i + 2 < NCH:
            for d in std[(i + 2) % ND]:
                d.wait()
            gd[(i + 2) % ND] = _gather(i + 2)
        gd[r].wait()

        def _scale(j, carry, r=r):
            wvec = wv[r][pl.ds(j * 16, 16)]
            for t in range(16):
                k = j * 16 + t
                rows[r][k] = rows[r][k] * wvec[t]
            return carry

        lax.fori_loop(0, KCM // 16, _scale, 0)
        sd[r] = pltpu.async_copy(rows[r], v_sh.at[idxs[r]], ssem[r],
                                 add=True)
    sd[(NCH - 1) % ND].wait()
    plsc.subcore_barrier()

    lanes = lax.iota(jnp.int32, 16)
    wod = []
    for j in range(NSL):
        buf = rows[j % ND]
        pltpu.sync_copy(v_sh.at[pl.ds(s * SLICE + j * KCM, KCM)], buf)
        for cls in range(CP):
            cvec = jnp.full((16,), cls, jnp.int32)

            def _t(q, carry, buf=buf, cls=cls, cvec=cvec):
                g = plsc.load_gather(buf, [q * 16 + lanes, cvec])
                tbuf[cls, pl.ds(q * 16, 16)] = g
                return carry

            lax.fori_loop(0, KCM // 16, _t, 0)
        for cls in range(CP):
            wod.append(pltpu.async_copy(
                tbuf.at[cls],
                v_out.at[c * CP + cls, pl.ds(s * SLICE + j * KCM, KCM)],
                swo))
        for d in wod:
            d.wait()
        wod = []


_msg_call = functools.partial(
    pl.kernel,
    out_type=jax.ShapeDtypeStruct((NC * CP, NP), jnp.float32),
    mesh=_mesh,
    scratch_types=(
        [pltpu.VMEM((KCM,), jnp.int32)] * 8
        + [pltpu.VMEM((KCM,), jnp.float32)] * 4
        + [pltpu.VMEM((KCM, CP), jnp.float32)] * 4
        + [pltpu.VMEM((CP, KCM), jnp.float32)]
        + [pltpu.SemaphoreType.DMA] * 13
        + [pltpu.VMEM_SHARED((NP, CP), jnp.float32)]
    ),
    compiler_params=pltpu.CompilerParams(use_tc_tiling_on_sc=False, needs_layout_passes=False),
)(_msg_body)


def _final_body(x_ref, vt_ref, fcw_ref, dis_ref, b_ref, out_ref, acc):
    k = pl.program_id(0)

    @pl.when(k == 0)
    def _init():
        acc[...] = jnp.zeros_like(acc)

    vt = vt_ref[...]
    dis = dis_ref[...]
    z = dis * (vt[0] + vt[1] + dis * fcw_ref[...])
    acc[...] += lax.dot_general(
        x_ref[...], z, (((1,), (1,)), ((), ())),
        precision=lax.Precision.HIGHEST,
        preferred_element_type=jnp.float32)

    @pl.when(k == NBLK - 1)
    def _fin():
        logits = acc[...] + b_ref[...]
        m = jnp.max(logits, axis=1, keepdims=True)
        e = jnp.exp(logits - m)
        out_ref[...] = (e / jnp.sum(e, axis=1, keepdims=True))[:, :C]


def _final_call(xp, vt3, fcw_pad, dis_row, bp):
    return pl.pallas_call(
        _final_body,
        grid=(NBLK,),
        in_specs=[
            pl.BlockSpec((B, BLK), lambda k: (0, k)),
            pl.BlockSpec((NC, CP, BLK), lambda k: (0, 0, k)),
            pl.BlockSpec((CP, BLK), lambda k: (0, k)),
            pl.BlockSpec((1, BLK), lambda k: (0, k)),
            pl.BlockSpec((1, CP), lambda k: (0, 0)),
        ],
        out_specs=pl.BlockSpec((B, C), lambda k: (0, 0)),
        out_shape=jax.ShapeDtypeStruct((B, C), jnp.float32),
        scratch_shapes=[pltpu.VMEM((B, CP), jnp.float32)],
    )(xp, vt3, fcw_pad, dis_row, bp)


def kernel(x, edge_index, edge_weight, fc_w, fc_b):
    ei3 = edge_index.astype(jnp.int32).reshape(2, E // 128, 128)
    ei3 = ei3.transpose(1, 0, 2)
    src = ei3[:, 0, :].reshape(E)
    dst = ei3[:, 1, :].reshape(E)
    pad_e = EP - E
    pad_idx = jnp.arange(pad_e, dtype=jnp.int32)
    src_p = jnp.concatenate([src, pad_idx])
    dst_p = jnp.concatenate([dst, pad_idx])
    w_p = jnp.concatenate([edge_weight, jnp.zeros((pad_e,), jnp.float32)])

    fcwt = jnp.pad(fc_w, ((0, CP - C), (0, NP - N))).T.reshape(NP * CP)
    xp = jnp.pad(x, ((0, 0), (0, NP - N)))
    bp = jnp.concatenate(
        [fc_b, jnp.full((CP - C,), -1e30, jnp.float32)]
    ).reshape(1, CP)

    deg_flat = _deg_call(dst_p, w_p)
    dis_lin, u_flat = _prep_call(deg_flat, fcwt)
    u = u_flat.reshape(NP, CP)
    src2 = src_p.reshape(EP // KCM, KCM)
    dst2 = dst_p.reshape(EP // KCM, KCM)
    w2 = w_p.reshape(EP // KCM, KCM)
    vt = _msg_call(src2, dst2, w2, u)
    vt3 = vt.reshape(NC, CP, NP)
    dis_row = dis_lin.reshape(1, NP)
    fcw_pad = jnp.pad(fc_w, ((0, CP - C), (0, NP - N)))
    return _final_call(xp, vt3, fcw_pad, dis_row, bp)
